# baseline (speedup 1.0000x reference)
import functools

import jax
import jax.numpy as jnp
from jax import lax
from jax.experimental import pallas as pl
from jax.experimental.pallas import tpu as pltpu
from jax.experimental.pallas import tpu_sc as plsc

_GAMMA = 12.0
_GAMMA_TYPE = 6.0
_HID = 64
_TDIM = 32
_CHUNK = 128


@functools.cache
def _build(B):
  info = plsc.get_sparse_core_info()
  NC, NS, L = info.num_cores, info.num_subcores, info.num_lanes
  NW = NC * NS
  assert B % (NW * _CHUNK) == 0
  per_w = B // NW
  n_chunks = per_w // _CHUNK
  groups = _CHUNK // L
  f32 = jnp.float32
  i32 = jnp.int32
  mesh = plsc.VectorSubcoreMesh(core_axis_name="c", subcore_axis_name="s")

  @functools.partial(
      pl.kernel,
      mesh=mesh,
      compiler_params=pltpu.CompilerParams(
          needs_layout_passes=False, use_tc_tiling_on_sc=False),
      out_type=[jax.ShapeDtypeStruct((B,), f32),
                jax.ShapeDtypeStruct((B,), f32)],
      scratch_types=[
          pltpu.VMEM((_CHUNK,), i32),
          pltpu.VMEM((_CHUNK,), i32),
          pltpu.VMEM((_CHUNK,), i32),
          pltpu.VMEM((_CHUNK, _HID), f32),
          pltpu.VMEM((_CHUNK, _HID), f32),
          pltpu.VMEM((_CHUNK, _HID), f32),
          pltpu.VMEM((_CHUNK, _HID), f32),
          pltpu.VMEM((_CHUNK, _TDIM), f32),
          pltpu.VMEM((_CHUNK, _TDIM), f32),
          pltpu.VMEM((_CHUNK, _TDIM), f32),
          pltpu.VMEM((_CHUNK, _TDIM), f32),
          pltpu.VMEM((_HID, 16), f32),
          pltpu.VMEM((_HID, 16), f32),
          pltpu.VMEM((_CHUNK,), f32),
          pltpu.VMEM((_CHUNK,), f32),
          pltpu.SemaphoreType.DMA,
      ],
  )
  def kge(hidx_hbm, ridx_hbm, tidx_hbm,
          ent_hbm, rel_hbm, typ_hbm, rtyp_hbm, nv_hbm, nvt_hbm,
          score_hbm, scoret_hbm,
          hidx_v, ridx_v, tidx_v,
          head_v, rel_v, tail_v, nv_v,
          htyp_v, rtyp_v, ttyp_v, nvt_v,
          a_scr, v_scr, sc_v, sct_v, sem):
    wid = lax.axis_index("s") * NC + lax.axis_index("c")
    base = wid * per_w

    def chunk_body(ci, carry):
      off = base + ci * _CHUNK
      pltpu.sync_copy(hidx_hbm.at[pl.ds(off, _CHUNK)], hidx_v)
      pltpu.sync_copy(ridx_hbm.at[pl.ds(off, _CHUNK)], ridx_v)
      pltpu.sync_copy(tidx_hbm.at[pl.ds(off, _CHUNK)], tidx_v)
      cps = [
          pltpu.async_copy(ent_hbm.at[hidx_v], head_v, sem),
          pltpu.async_copy(ent_hbm.at[tidx_v], tail_v, sem),
          pltpu.async_copy(rel_hbm.at[ridx_v], rel_v, sem),
          pltpu.async_copy(nv_hbm.at[ridx_v], nv_v, sem),
          pltpu.async_copy(typ_hbm.at[hidx_v], htyp_v, sem),
          pltpu.async_copy(typ_hbm.at[tidx_v], ttyp_v, sem),
          pltpu.async_copy(rtyp_hbm.at[ridx_v], rtyp_v, sem),
          pltpu.async_copy(nvt_hbm.at[ridx_v], nvt_v, sem),
      ]
      for cp in cps:
        cp.wait()

      def group_body(g, carry2):
        ids = g * L + lax.iota(i32, L)

        def project_score(hr, rr, tr, vr, D, gamma):
          hv = jnp.zeros((L,), f32)
          tv = jnp.zeros((L,), f32)
          vv = jnp.zeros((L,), f32)
          for d in range(D):
            col = jnp.full((L,), d, i32)
            h = plsc.load_gather(hr, [ids, col])
            r = plsc.load_gather(rr, [ids, col])
            t = plsc.load_gather(tr, [ids, col])
            v = plsc.load_gather(vr, [ids, col])
            hv = hv + h * v
            tv = tv + t * v
            vv = vv + v * v
            a_scr[d] = h + r - t
            v_scr[d] = v
          c = (tv - hv) / jnp.maximum(vv, 1e-24)
          acc = jnp.zeros((L,), f32)
          for d in range(D):
            acc = acc + jnp.abs(a_scr[d] + c * v_scr[d])
          return gamma - acc

        s = project_score(head_v, rel_v, tail_v, nv_v, _HID, _GAMMA)
        st = project_score(htyp_v, rtyp_v, ttyp_v, nvt_v, _TDIM, _GAMMA_TYPE)
        sc_v[pl.ds(g * L, L)] = s
        sct_v[pl.ds(g * L, L)] = st
        return carry2

      lax.fori_loop(0, groups, group_body, 0)
      pltpu.sync_copy(sc_v, score_hbm.at[pl.ds(off, _CHUNK)])
      pltpu.sync_copy(sct_v, scoret_hbm.at[pl.ds(off, _CHUNK)])
      return carry

    lax.fori_loop(0, n_chunks, chunk_body, 0)

  return kge


def kernel(sample, entity_embedding, relation_embedding, type_embedding,
           reltype_embedding, norm_vector_embedding, norm_vectortype_embedding):
  B = sample.shape[0]
  fn = _build(B)
  h_idx = sample[:, 0]
  r_idx = sample[:, 1]
  t_idx = sample[:, 2]
  score, score_type = fn(h_idx, r_idx, t_idx, entity_embedding,
                         relation_embedding, type_embedding,
                         reltype_embedding, norm_vector_embedding,
                         norm_vectortype_embedding)
  return score.reshape(B, 1), score_type.reshape(B, 1)


# speedup vs baseline: 1.0016x; 1.0016x over previous
import functools

import jax
import jax.numpy as jnp
from jax import lax
from jax.experimental import pallas as pl
from jax.experimental.pallas import tpu as pltpu
from jax.experimental.pallas import tpu_sc as plsc

_GAMMA = 12.0
_GAMMA_TYPE = 6.0
_HID = 64
_TDIM = 32
_CHUNK = 128
_NACC = 4


@functools.cache
def _build(B):
  info = plsc.get_sparse_core_info()
  NC, NS, L = info.num_cores, info.num_subcores, info.num_lanes
  NW = NC * NS
  assert B % (NW * _CHUNK) == 0
  per_w = B // NW
  n_chunks = per_w // _CHUNK
  assert n_chunks % 2 == 0
  groups = _CHUNK // L
  f32 = jnp.float32
  i32 = jnp.int32
  mesh = plsc.VectorSubcoreMesh(core_axis_name="c", subcore_axis_name="s")

  def _gather_set():
    return [
        pltpu.VMEM((_CHUNK,), i32),
        pltpu.VMEM((_CHUNK,), i32),
        pltpu.VMEM((_CHUNK,), i32),
        pltpu.VMEM((_CHUNK, _HID), f32),
        pltpu.VMEM((_CHUNK, _HID), f32),
        pltpu.VMEM((_CHUNK, _HID), f32),
        pltpu.VMEM((_CHUNK, _HID), f32),
        pltpu.VMEM((_CHUNK, _TDIM), f32),
        pltpu.VMEM((_CHUNK, _TDIM), f32),
        pltpu.VMEM((_CHUNK, _TDIM), f32),
        pltpu.VMEM((_CHUNK, _TDIM), f32),
        pltpu.SemaphoreType.DMA,
    ]

  @functools.partial(
      pl.kernel,
      mesh=mesh,
      compiler_params=pltpu.CompilerParams(
          needs_layout_passes=False,
          use_tc_tiling_on_sc=False,
          disable_bounds_checks=True,
      ),
      out_type=[jax.ShapeDtypeStruct((B,), f32),
                jax.ShapeDtypeStruct((B,), f32)],
      scratch_types=(
          [pltpu.VMEM((_CHUNK, 3), i32)]
          + _gather_set() + _gather_set()
          + [
              pltpu.VMEM((_HID, 16), f32),
              pltpu.VMEM((_HID, 16), f32),
              pltpu.VMEM((_CHUNK,), f32),
              pltpu.VMEM((_CHUNK,), f32),
          ]),
  )
  def kge(sample_hbm,
          ent_hbm, rel_hbm, typ_hbm, rtyp_hbm, nv_hbm, nvt_hbm,
          score_hbm, scoret_hbm,
          samp_v, *scratch):
    set0 = scratch[0:12]
    set1 = scratch[12:24]
    a_scr, v_scr, sc_v, sct_v = scratch[24:28]
    wid = lax.axis_index("s") * NC + lax.axis_index("c")
    base = wid * per_w

    def copies(bufs, off):
      hidx_v, ridx_v, tidx_v = bufs[0:3]
      head_v, rel_v, tail_v, nv_v, htyp_v, rtyp_v, ttyp_v, nvt_v = bufs[3:11]
      sem = bufs[11]
      return [
          pltpu.make_async_copy(ent_hbm.at[hidx_v], head_v, sem),
          pltpu.make_async_copy(ent_hbm.at[tidx_v], tail_v, sem),
          pltpu.make_async_copy(rel_hbm.at[ridx_v], rel_v, sem),
          pltpu.make_async_copy(nv_hbm.at[ridx_v], nv_v, sem),
          pltpu.make_async_copy(typ_hbm.at[hidx_v], htyp_v, sem),
          pltpu.make_async_copy(typ_hbm.at[tidx_v], ttyp_v, sem),
          pltpu.make_async_copy(rtyp_hbm.at[ridx_v], rtyp_v, sem),
          pltpu.make_async_copy(nvt_hbm.at[ridx_v], nvt_v, sem),
      ]

    def start_chunk(bufs, ci):
      off = base + ci * _CHUNK
      hidx_v, ridx_v, tidx_v = bufs[0:3]
      pltpu.sync_copy(sample_hbm.at[pl.ds(off, _CHUNK)], samp_v)
      col0 = jnp.zeros((L,), i32)
      for g in range(groups):
        ids = g * L + lax.iota(i32, L)
        hidx_v[pl.ds(g * L, L)] = plsc.load_gather(samp_v, [ids, col0])
        ridx_v[pl.ds(g * L, L)] = plsc.load_gather(samp_v, [ids, col0 + 1])
        tidx_v[pl.ds(g * L, L)] = plsc.load_gather(samp_v, [ids, col0 + 2])
      for cp in copies(bufs, ci):
        cp.start()

    def wait_chunk(bufs, ci):
      for cp in copies(bufs, ci):
        cp.wait()

    def compute_chunk(bufs, ci):
      off = base + ci * _CHUNK
      head_v, rel_v, tail_v, nv_v, htyp_v, rtyp_v, ttyp_v, nvt_v = bufs[3:11]

      def group_body(g, carry):
        ids = g * L + lax.iota(i32, L)

        def project_score(hr, rr, tr, vr, D, gamma):
          hvs = [jnp.zeros((L,), f32) for _ in range(_NACC)]
          tvs = [jnp.zeros((L,), f32) for _ in range(_NACC)]
          vvs = [jnp.zeros((L,), f32) for _ in range(_NACC)]
          for d in range(D):
            j = d % _NACC
            col = jnp.full((L,), d, i32)
            h = plsc.load_gather(hr, [ids, col])
            r = plsc.load_gather(rr, [ids, col])
            t = plsc.load_gather(tr, [ids, col])
            v = plsc.load_gather(vr, [ids, col])
            hvs[j] = hvs[j] + h * v
            tvs[j] = tvs[j] + t * v
            vvs[j] = vvs[j] + v * v
            a_scr[d] = h + r - t
            v_scr[d] = v
          hv = (hvs[0] + hvs[1]) + (hvs[2] + hvs[3])
          tv = (tvs[0] + tvs[1]) + (tvs[2] + tvs[3])
          vv = (vvs[0] + vvs[1]) + (vvs[2] + vvs[3])
          c = (tv - hv) / jnp.maximum(vv, 1e-24)
          accs = [jnp.zeros((L,), f32) for _ in range(_NACC)]
          for d in range(D):
            accs[d % _NACC] = accs[d % _NACC] + jnp.abs(a_scr[d] + c * v_scr[d])
          return gamma - ((accs[0] + accs[1]) + (accs[2] + accs[3]))

        s = project_score(head_v, rel_v, tail_v, nv_v, _HID, _GAMMA)
        st = project_score(htyp_v, rtyp_v, ttyp_v, nvt_v, _TDIM, _GAMMA_TYPE)
        sc_v[pl.ds(g * L, L)] = s
        sct_v[pl.ds(g * L, L)] = st
        return carry

      lax.fori_loop(0, groups, group_body, 0)
      pltpu.sync_copy(sc_v, score_hbm.at[pl.ds(off, _CHUNK)])
      pltpu.sync_copy(sct_v, scoret_hbm.at[pl.ds(off, _CHUNK)])

    start_chunk(set0, 0)

    def chunk_pair(ci2, carry):
      ci = ci2 * 2
      wait_chunk(set0, ci)
      start_chunk(set1, ci + 1)
      compute_chunk(set0, ci)
      wait_chunk(set1, ci + 1)

      @pl.when(ci + 2 < n_chunks)
      def _():
        start_chunk(set0, ci + 2)

      compute_chunk(set1, ci + 1)
      return carry

    lax.fori_loop(0, n_chunks // 2, chunk_pair, 0)

  return kge


def kernel(sample, entity_embedding, relation_embedding, type_embedding,
           reltype_embedding, norm_vector_embedding, norm_vectortype_embedding):
  B = sample.shape[0]
  fn = _build(B)
  score, score_type = fn(sample, entity_embedding,
                         relation_embedding, type_embedding,
                         reltype_embedding, norm_vector_embedding,
                         norm_vectortype_embedding)
  return score.reshape(B, 1), score_type.reshape(B, 1)


# speedup vs baseline: 1.2798x; 1.2777x over previous
import functools

import jax
import jax.numpy as jnp
from jax import lax
from jax.experimental import pallas as pl
from jax.experimental.pallas import tpu as pltpu
from jax.experimental.pallas import tpu_sc as plsc

_GAMMA = 12.0
_GAMMA_TYPE = 6.0
_HID = 64
_TDIM = 32
_C = 32
_NIDX = 9


@functools.cache
def _build(B, E2, R2, E4, R4):
  info = plsc.get_sparse_core_info()
  NC, NS, L = info.num_cores, info.num_subcores, info.num_lanes
  NW = NC * NS
  assert B % (NW * _C) == 0
  per_w = B // NW
  n_chunks = per_w // _C
  assert n_chunks % 2 == 0
  groups = _C // L
  f32 = jnp.float32
  i32 = jnp.int32
  mesh = plsc.VectorSubcoreMesh(core_axis_name="c", subcore_axis_name="s")

  def _set():
    return [
        pltpu.VMEM((_NIDX * _C,), i32),
        pltpu.VMEM((2 * _C, 128), f32),
        pltpu.VMEM((2 * _C, 128), f32),
        pltpu.VMEM((_C, 128), f32),
        pltpu.VMEM((_C, 128), f32),
        pltpu.VMEM((_C, 128), f32),
        pltpu.VMEM((_C, 128), f32),
        pltpu.SemaphoreType.DMA,
    ]

  @functools.partial(
      pl.kernel,
      mesh=mesh,
      compiler_params=pltpu.CompilerParams(
          needs_layout_passes=False,
          disable_bounds_checks=True,
      ),
      out_type=[jax.ShapeDtypeStruct((B,), f32),
                jax.ShapeDtypeStruct((B,), f32)],
      scratch_types=(
          _set() + _set()
          + [
              pltpu.VMEM((per_w,), f32),
              pltpu.VMEM((per_w,), f32),
          ]),
  )
  def kge(pack_hbm, ent_hbm, rel_hbm, typ_hbm, rtyp_hbm, nv_hbm, nvt_hbm,
          score_hbm, scoret_hbm, *scratch):
    set0 = scratch[0:8]
    set1 = scratch[8:16]
    sc_v, sct_v = scratch[16:18]
    wid = lax.axis_index("s") * NC + lax.axis_index("c")
    base = wid * per_w

    def copies(bufs):
      idx_v, ent_v, typ_v, rel_v, nv_v, rtyp_v, nvt_v, sem = bufs
      return [
          pltpu.make_async_copy(
              ent_hbm.at[idx_v.at[pl.ds(0, 2 * _C)]], ent_v, sem),
          pltpu.make_async_copy(
              typ_hbm.at[idx_v.at[pl.ds(2 * _C, 2 * _C)]], typ_v, sem),
          pltpu.make_async_copy(
              rel_hbm.at[idx_v.at[pl.ds(4 * _C, _C)]], rel_v, sem),
          pltpu.make_async_copy(
              nv_hbm.at[idx_v.at[pl.ds(4 * _C, _C)]], nv_v, sem),
          pltpu.make_async_copy(
              rtyp_hbm.at[idx_v.at[pl.ds(5 * _C, _C)]], rtyp_v, sem),
          pltpu.make_async_copy(
              nvt_hbm.at[idx_v.at[pl.ds(5 * _C, _C)]], nvt_v, sem),
      ]

    def start_chunk(bufs, ci):
      idx_v = bufs[0]
      gchunk = wid * n_chunks + ci
      pltpu.sync_copy(pack_hbm.at[pl.ds(gchunk * (_NIDX * _C), _NIDX * _C)],
                      idx_v)
      for cp in copies(bufs):
        cp.start()

    def wait_chunk(bufs):
      for cp in copies(bufs):
        cp.wait()

    def compute_chunk(bufs, ci):
      idx_v, ent_v, typ_v, rel_v, nv_v, rtyp_v, nvt_v, _ = bufs
      lane = lax.iota(i32, L)

      def rsum(x):
        return jnp.broadcast_to(jnp.sum(x), (L,))

      def group_body(g, carry):
        score_acc = jnp.zeros((L,), f32)
        scoret_acc = jnp.zeros((L,), f32)
        hvec = idx_v[pl.ds(6 * _C + g * L, L)]
        rvec = idx_v[pl.ds(7 * _C + g * L, L)]
        tvec = idx_v[pl.ds(8 * _C + g * L, L)]
        for k in range(L):
          i = g * L + k
          h = hvec[k]
          r = rvec[k]
          t = tvec[k]
          hoff = (h & 1) << 6
          roff = (r & 1) << 6
          toff = (t & 1) << 6
          h4o = (h & 3) << 5
          r4o = (r & 3) << 5
          t4o = (t & 3) << 5

          hs = [ent_v[i, pl.ds(hoff + 16 * q, 16)] for q in range(4)]
          ts = [ent_v[_C + i, pl.ds(toff + 16 * q, 16)] for q in range(4)]
          rs = [rel_v[i, pl.ds(roff + 16 * q, 16)] for q in range(4)]
          vs = [nv_v[i, pl.ds(roff + 16 * q, 16)] for q in range(4)]
          hv = rsum((hs[0] * vs[0] + hs[1] * vs[1])
                    + (hs[2] * vs[2] + hs[3] * vs[3]))
          tv = rsum((ts[0] * vs[0] + ts[1] * vs[1])
                    + (ts[2] * vs[2] + ts[3] * vs[3]))
          vv = rsum((vs[0] * vs[0] + vs[1] * vs[1])
                    + (vs[2] * vs[2] + vs[3] * vs[3]))
          c = (tv - hv) / jnp.maximum(vv, 1e-24)
          s4 = [jnp.abs(hs[q] + rs[q] - ts[q] + c * vs[q]) for q in range(4)]
          score = _GAMMA - rsum((s4[0] + s4[1]) + (s4[2] + s4[3]))

          h2s = [typ_v[i, pl.ds(h4o + 16 * q, 16)] for q in range(2)]
          t2s = [typ_v[_C + i, pl.ds(t4o + 16 * q, 16)] for q in range(2)]
          r2s = [rtyp_v[i, pl.ds(r4o + 16 * q, 16)] for q in range(2)]
          v2s = [nvt_v[i, pl.ds(r4o + 16 * q, 16)] for q in range(2)]
          hv2 = rsum(h2s[0] * v2s[0] + h2s[1] * v2s[1])
          tv2 = rsum(t2s[0] * v2s[0] + t2s[1] * v2s[1])
          vv2 = rsum(v2s[0] * v2s[0] + v2s[1] * v2s[1])
          c2 = (tv2 - hv2) / jnp.maximum(vv2, 1e-24)
          s2 = [jnp.abs(h2s[q] + r2s[q] - t2s[q] + c2 * v2s[q])
                for q in range(2)]
          score_t = _GAMMA_TYPE - rsum(s2[0] + s2[1])

          score_acc = jnp.where(lane == k, score, score_acc)
          scoret_acc = jnp.where(lane == k, score_t, scoret_acc)

        out_off = ci * _C + g * L
        sc_v[pl.ds(out_off, L)] = score_acc
        sct_v[pl.ds(out_off, L)] = scoret_acc
        return carry

      lax.fori_loop(0, groups, group_body, 0)

    start_chunk(set0, 0)

    def chunk_pair(ci2, carry):
      ci = ci2 * 2
      wait_chunk(set0)
      start_chunk(set1, ci + 1)
      compute_chunk(set0, ci)
      wait_chunk(set1)

      @pl.when(ci + 2 < n_chunks)
      def _():
        start_chunk(set0, ci + 2)

      compute_chunk(set1, ci + 1)
      return carry

    lax.fori_loop(0, n_chunks // 2, chunk_pair, 0)
    pltpu.sync_copy(sc_v, score_hbm.at[pl.ds(base, per_w)])
    pltpu.sync_copy(sct_v, scoret_hbm.at[pl.ds(base, per_w)])

  return kge


def kernel(sample, entity_embedding, relation_embedding, type_embedding,
           reltype_embedding, norm_vector_embedding, norm_vectortype_embedding):
  B = sample.shape[0]
  E, _ = entity_embedding.shape
  R, _ = relation_embedding.shape
  fn = _build(B, E // 2, R // 2, E // 4, R // 4)
  h = sample[:, 0]
  r = sample[:, 1]
  t = sample[:, 2]
  idx9 = jnp.stack([h >> 1, t >> 1, h >> 2, t >> 2, r >> 1, r >> 2, h, r, t])
  pack = idx9.reshape(_NIDX, B // _C, _C).transpose(1, 0, 2).reshape(-1)
  score, score_type = fn(
      pack,
      entity_embedding.reshape(E // 2, 2 * _HID),
      relation_embedding.reshape(R // 2, 2 * _HID),
      type_embedding.reshape(E // 4, 4 * _TDIM),
      reltype_embedding.reshape(R // 4, 4 * _TDIM),
      norm_vector_embedding.reshape(R // 2, 2 * _HID),
      norm_vectortype_embedding.reshape(R // 4, 4 * _TDIM),
  )
  return score.reshape(B, 1), score_type.reshape(B, 1)


# speedup vs baseline: 1.3532x; 1.0574x over previous
import functools

import jax
import jax.numpy as jnp
from jax import lax
from jax.experimental import pallas as pl
from jax.experimental.pallas import tpu as pltpu
from jax.experimental.pallas import tpu_sc as plsc

_GAMMA = 12.0
_GAMMA_TYPE = 6.0
_HID = 64
_TDIM = 32
_C = 128
_NIDX = 3


@functools.cache
def _build(B):
  info = plsc.get_sparse_core_info()
  NC, NS, L = info.num_cores, info.num_subcores, info.num_lanes
  NW = NC * NS
  assert B % (NW * _C) == 0
  per_w = B // NW
  n_chunks = per_w // _C
  assert n_chunks % 2 == 0
  groups = _C // L
  f32 = jnp.float32
  i32 = jnp.int32
  mesh = plsc.VectorSubcoreMesh(core_axis_name="c", subcore_axis_name="s")

  def _set():
    return [
        pltpu.VMEM((_NIDX * _C,), i32),
        pltpu.VMEM((2 * _C, _HID), f32),
        pltpu.VMEM((2 * _C, _TDIM), f32),
        pltpu.VMEM((_C, _HID), f32),
        pltpu.VMEM((_C, _HID), f32),
        pltpu.VMEM((_C, _TDIM), f32),
        pltpu.VMEM((_C, _TDIM), f32),
        pltpu.SemaphoreType.DMA,
    ]

  @functools.partial(
      pl.kernel,
      mesh=mesh,
      compiler_params=pltpu.CompilerParams(
          needs_layout_passes=False,
          use_tc_tiling_on_sc=False,
          disable_bounds_checks=True,
      ),
      out_type=[jax.ShapeDtypeStruct((B,), f32),
                jax.ShapeDtypeStruct((B,), f32)],
      scratch_types=(
          _set() + _set()
          + [
              pltpu.VMEM((per_w,), f32),
              pltpu.VMEM((per_w,), f32),
          ]),
  )
  def kge(pack_hbm, ent_hbm, rel_hbm, typ_hbm, rtyp_hbm, nv_hbm, nvt_hbm,
          score_hbm, scoret_hbm, *scratch):
    set0 = scratch[0:8]
    set1 = scratch[8:16]
    sc_v, sct_v = scratch[16:18]
    wid = lax.axis_index("s") * NC + lax.axis_index("c")
    base = wid * per_w

    def copies(bufs):
      idx_v, ent_v, typ_v, rel_v, nv_v, rtyp_v, nvt_v, sem = bufs
      ht = idx_v.at[pl.ds(0, 2 * _C)]
      rr = idx_v.at[pl.ds(2 * _C, _C)]
      return [
          pltpu.make_async_copy(ent_hbm.at[ht], ent_v, sem),
          pltpu.make_async_copy(typ_hbm.at[ht], typ_v, sem),
          pltpu.make_async_copy(rel_hbm.at[rr], rel_v, sem),
          pltpu.make_async_copy(nv_hbm.at[rr], nv_v, sem),
          pltpu.make_async_copy(rtyp_hbm.at[rr], rtyp_v, sem),
          pltpu.make_async_copy(nvt_hbm.at[rr], nvt_v, sem),
      ]

    def start_chunk(bufs, ci):
      idx_v = bufs[0]
      gchunk = wid * n_chunks + ci
      pltpu.sync_copy(pack_hbm.at[pl.ds(gchunk * (_NIDX * _C), _NIDX * _C)],
                      idx_v)
      for cp in copies(bufs):
        cp.start()

    def wait_chunk(bufs):
      for cp in copies(bufs):
        cp.wait()

    def compute_chunk(bufs, ci):
      idx_v, ent_v, typ_v, rel_v, nv_v, rtyp_v, nvt_v, _ = bufs
      lane = lax.iota(i32, L)

      def rsum(x):
        return jnp.broadcast_to(jnp.sum(x), (L,))

      def group_body(g, carry):
        score_acc = jnp.zeros((L,), f32)
        scoret_acc = jnp.zeros((L,), f32)
        for k in range(L):
          i = g * L + k

          hs = [ent_v[i, pl.ds(16 * q, 16)] for q in range(4)]
          ts = [ent_v[_C + i, pl.ds(16 * q, 16)] for q in range(4)]
          rs = [rel_v[i, pl.ds(16 * q, 16)] for q in range(4)]
          vs = [nv_v[i, pl.ds(16 * q, 16)] for q in range(4)]
          hv = rsum((hs[0] * vs[0] + hs[1] * vs[1])
                    + (hs[2] * vs[2] + hs[3] * vs[3]))
          tv = rsum((ts[0] * vs[0] + ts[1] * vs[1])
                    + (ts[2] * vs[2] + ts[3] * vs[3]))
          vv = rsum((vs[0] * vs[0] + vs[1] * vs[1])
                    + (vs[2] * vs[2] + vs[3] * vs[3]))
          c = (tv - hv) / jnp.maximum(vv, 1e-24)
          s4 = [jnp.abs(hs[q] + rs[q] - ts[q] + c * vs[q]) for q in range(4)]
          score = _GAMMA - rsum((s4[0] + s4[1]) + (s4[2] + s4[3]))

          h2s = [typ_v[i, pl.ds(16 * q, 16)] for q in range(2)]
          t2s = [typ_v[_C + i, pl.ds(16 * q, 16)] for q in range(2)]
          r2s = [rtyp_v[i, pl.ds(16 * q, 16)] for q in range(2)]
          v2s = [nvt_v[i, pl.ds(16 * q, 16)] for q in range(2)]
          hv2 = rsum(h2s[0] * v2s[0] + h2s[1] * v2s[1])
          tv2 = rsum(t2s[0] * v2s[0] + t2s[1] * v2s[1])
          vv2 = rsum(v2s[0] * v2s[0] + v2s[1] * v2s[1])
          c2 = (tv2 - hv2) / jnp.maximum(vv2, 1e-24)
          s2 = [jnp.abs(h2s[q] + r2s[q] - t2s[q] + c2 * v2s[q])
                for q in range(2)]
          score_t = _GAMMA_TYPE - rsum(s2[0] + s2[1])

          score_acc = jnp.where(lane == k, score, score_acc)
          scoret_acc = jnp.where(lane == k, score_t, scoret_acc)

        out_off = ci * _C + g * L
        sc_v[pl.ds(out_off, L)] = score_acc
        sct_v[pl.ds(out_off, L)] = scoret_acc
        return carry

      lax.fori_loop(0, groups, group_body, 0)

    start_chunk(set0, 0)

    def chunk_pair(ci2, carry):
      ci = ci2 * 2
      wait_chunk(set0)
      start_chunk(set1, ci + 1)
      compute_chunk(set0, ci)
      wait_chunk(set1)

      @pl.when(ci + 2 < n_chunks)
      def _():
        start_chunk(set0, ci + 2)

      compute_chunk(set1, ci + 1)
      return carry

    lax.fori_loop(0, n_chunks // 2, chunk_pair, 0)
    pltpu.sync_copy(sc_v, score_hbm.at[pl.ds(base, per_w)])
    pltpu.sync_copy(sct_v, scoret_hbm.at[pl.ds(base, per_w)])

  return kge


def kernel(sample, entity_embedding, relation_embedding, type_embedding,
           reltype_embedding, norm_vector_embedding, norm_vectortype_embedding):
  B = sample.shape[0]
  fn = _build(B)
  h = sample[:, 0]
  r = sample[:, 1]
  t = sample[:, 2]
  idx3 = jnp.stack([h, t, r])
  pack = idx3.reshape(_NIDX, B // _C, _C).transpose(1, 0, 2).reshape(-1)
  score, score_type = fn(
      pack, entity_embedding, relation_embedding, type_embedding,
      reltype_embedding, norm_vector_embedding, norm_vectortype_embedding)
  return score.reshape(B, 1), score_type.reshape(B, 1)
